# Initial kernel scaffold; baseline (speedup 1.0000x reference)
import functools

import jax
import jax.numpy as jnp
from jax import lax
from jax.experimental import pallas as pl
from jax.experimental.pallas import tpu as pltpu
from jax.experimental.pallas import tpu_sc as plsc

NC = 2
NS = 16
CH = 128


def _sc_segment_sums(x, row, col, eattr, zx, ze):
    N, D = x.shape
    E, DE = eattr.shape
    n_chunks = E // CH
    nw = NC * NS
    base_chunks = n_chunks // nw
    extra = n_chunks - base_chunks * nw

    mesh = plsc.VectorSubcoreMesh(core_axis_name="c", subcore_axis_name="s")

    @functools.partial(
        pl.kernel,
        out_type=(
            jax.ShapeDtypeStruct((NC, N, D), jnp.float32),
            jax.ShapeDtypeStruct((NC, N, DE), jnp.float32),
            jax.ShapeDtypeStruct((NC, N, DE), jnp.float32),
        ),
        mesh=mesh,
        scratch_types=[
            pltpu.VMEM((CH,), jnp.int32),
            pltpu.VMEM((CH,), jnp.int32),
            pltpu.VMEM((CH, D), jnp.float32),
            pltpu.VMEM((CH, DE), jnp.float32),
            pltpu.VMEM((CH, DE), jnp.float32),
            pltpu.VMEM_SHARED((N, D), jnp.float32),
            pltpu.VMEM_SHARED((N, DE), jnp.float32),
            pltpu.VMEM_SHARED((N, DE), jnp.float32),
            pltpu.SemaphoreType.DMA,
        ],
    )
    def sc_kernel(x_hbm, row_hbm, col_hbm, ea_hbm, zx_hbm, ze_hbm,
                  ox_hbm, oe_hbm, oc_hbm,
                  ridx_v, cidx_v, msg_v, ea_v, ones_v,
                  accx_s, acce_s, accc_s, sem):
        cid = lax.axis_index("c")
        sid = lax.axis_index("s")
        gw = cid * NS + sid

        @pl.loop(0, CH)
        def _(i):
            ones_v[i, :] = jnp.full((DE,), 1.0, dtype=jnp.float32)

        rpt = N // NS
        zbase = sid * rpt
        pltpu.async_copy(zx_hbm.at[pl.ds(zbase, rpt)],
                         accx_s.at[pl.ds(zbase, rpt)], sem).wait()
        pltpu.async_copy(ze_hbm.at[pl.ds(zbase, rpt)],
                         acce_s.at[pl.ds(zbase, rpt)], sem).wait()
        pltpu.async_copy(ze_hbm.at[pl.ds(zbase, rpt)],
                         accc_s.at[pl.ds(zbase, rpt)], sem).wait()
        plsc.subcore_barrier()

        def do_chunk(c):
            ebase = c * CH
            pltpu.sync_copy(row_hbm.at[pl.ds(ebase, CH)], ridx_v)
            pltpu.sync_copy(col_hbm.at[pl.ds(ebase, CH)], cidx_v)
            pltpu.sync_copy(ea_hbm.at[pl.ds(ebase, CH)], ea_v)
            pltpu.async_copy(x_hbm.at[ridx_v], msg_v, sem).wait()
            pltpu.sync_copy(msg_v, accx_s.at[cidx_v], add=True)
            pltpu.sync_copy(ea_v, acce_s.at[cidx_v], add=True)
            pltpu.sync_copy(ones_v, accc_s.at[cidx_v], add=True)

        @pl.loop(0, base_chunks)
        def _(i):
            do_chunk(i * nw + gw)

        if extra:
            @pl.when(gw < extra)
            def _():
                do_chunk(base_chunks * nw + gw)

        plsc.subcore_barrier()

        pltpu.sync_copy(accx_s.at[pl.ds(zbase, rpt)],
                        ox_hbm.at[cid, pl.ds(zbase, rpt)])
        pltpu.sync_copy(acce_s.at[pl.ds(zbase, rpt)],
                        oe_hbm.at[cid, pl.ds(zbase, rpt)])
        pltpu.sync_copy(accc_s.at[pl.ds(zbase, rpt)],
                        oc_hbm.at[cid, pl.ds(zbase, rpt)])

    return sc_kernel(x, row, col, eattr, zx, ze)


def _tc_body(x_ref, sx0_ref, sx1_ref, se0_ref, se1_ref, c0_ref, c1_ref,
             w_ref, b_ref, o_ref):
    D = x_ref.shape[1]
    DE = se0_ref.shape[2]
    sx = sx0_ref[0] + sx1_ref[0]
    se = se0_ref[0] + se1_ref[0]
    cnt = c0_ref[0][:, 0:1] + c1_ref[0][:, 0:1]
    inv = 1.0 / jnp.maximum(cnt, 1.0)
    acc = jnp.dot(x_ref[...], w_ref[0:D, :], preferred_element_type=jnp.float32)
    agg = jnp.dot(sx, w_ref[D:2 * D, :], preferred_element_type=jnp.float32)
    agg = agg + jnp.dot(se, w_ref[2 * D:2 * D + DE, :],
                        preferred_element_type=jnp.float32)
    o_ref[...] = jnp.maximum(acc + agg * inv + b_ref[...], 0.0)


def _tc_update(x, ox, oe, oc, W, b):
    N, D = x.shape
    DE = oe.shape[2]
    BN = 1000
    grid = (N // BN,)
    fan_in = W.shape[0]
    return pl.pallas_call(
        _tc_body,
        grid=grid,
        in_specs=[
            pl.BlockSpec((BN, D), lambda i: (i, 0)),
            pl.BlockSpec((1, BN, D), lambda i: (0, i, 0)),
            pl.BlockSpec((1, BN, D), lambda i: (1, i, 0)),
            pl.BlockSpec((1, BN, DE), lambda i: (0, i, 0)),
            pl.BlockSpec((1, BN, DE), lambda i: (1, i, 0)),
            pl.BlockSpec((1, BN, DE), lambda i: (0, i, 0)),
            pl.BlockSpec((1, BN, DE), lambda i: (1, i, 0)),
            pl.BlockSpec((fan_in, D), lambda i: (0, 0)),
            pl.BlockSpec((1, D), lambda i: (0, 0)),
        ],
        out_specs=pl.BlockSpec((BN, D), lambda i: (i, 0)),
        out_shape=jax.ShapeDtypeStruct((N, D), jnp.float32),
    )(x, ox, ox, oe, oe, oc, oc, W, b.reshape(1, D))


def kernel(x, edge_index, edge_attr, W, b):
    N, D = x.shape
    DE = edge_attr.shape[1]
    row = edge_index[0]
    col = edge_index[1]
    zx = jnp.zeros((N, D), jnp.float32)
    ze = jnp.zeros((N, DE), jnp.float32)
    ox, oe, oc = _sc_segment_sums(x, row, col, edge_attr, zx, ze)
    x_new = _tc_update(x, ox, oe, oc, W, b)
    return (x_new, edge_attr)


# speedup vs baseline: 3.9301x; 3.9301x over previous
import functools

import jax
import jax.numpy as jnp
from jax import lax
from jax.experimental import pallas as pl
from jax.experimental.pallas import tpu as pltpu
from jax.experimental.pallas import tpu_sc as plsc

NC = 2
NS = 16
CH = 128


def _sc_segment_sums(xlo, xhi, row, col, eattr, zx, ze):
    N, DH = xlo.shape
    E, DE = eattr.shape
    n_chunks = E // CH
    base_chunks = n_chunks // NS
    extra = n_chunks - base_chunks * NS

    mesh = plsc.VectorSubcoreMesh(core_axis_name="c", subcore_axis_name="s")

    @functools.partial(
        pl.kernel,
        out_type=(
            jax.ShapeDtypeStruct((NC, N, DH), jnp.float32),
            jax.ShapeDtypeStruct((NC, N, DE), jnp.float32),
        ),
        mesh=mesh,
        compiler_params=pltpu.CompilerParams(use_tc_tiling_on_sc=False),
        scratch_types=[
            pltpu.VMEM((CH,), jnp.int32),
            pltpu.VMEM((CH,), jnp.int32),
            pltpu.VMEM((CH, DH), jnp.float32),
            pltpu.VMEM((CH, DE), jnp.float32),
            pltpu.VMEM((CH, DE), jnp.float32),
            pltpu.VMEM_SHARED((N, DH), jnp.float32),
            pltpu.VMEM_SHARED((N, DE), jnp.float32),
            pltpu.SemaphoreType.DMA,
        ],
    )
    def sc_kernel(xlo_hbm, xhi_hbm, row_hbm, col_hbm, ea_hbm, zx_hbm, ze_hbm,
                  ox_hbm, o2_hbm,
                  ridx_v, cidx_v, msg_v, ea_v, ones_v,
                  accx_s, acc2_s, sem):
        cid = lax.axis_index("c")
        sid = lax.axis_index("s")

        @pl.loop(0, CH)
        def _(i):
            ones_v[i, :] = jnp.full((DE,), 1.0, dtype=jnp.float32)

        rpt = (N // NS) // 8 * 8
        rem = N - NS * rpt
        zbase = sid * rpt

        def zero_rows(base, size):
            pltpu.async_copy(zx_hbm.at[pl.ds(base, size)],
                             accx_s.at[pl.ds(base, size)], sem).wait()
            pltpu.async_copy(ze_hbm.at[pl.ds(base, size)],
                             acc2_s.at[pl.ds(base, size)], sem).wait()

        zero_rows(zbase, rpt)
        if rem:
            @pl.when(sid == NS - 1)
            def _():
                zero_rows(NS * rpt, rem)
        plsc.subcore_barrier()

        def do_chunk(c):
            ebase = c * CH
            pltpu.sync_copy(row_hbm.at[pl.ds(ebase, CH)], ridx_v)
            pltpu.sync_copy(col_hbm.at[pl.ds(ebase, CH)], cidx_v)

            @pl.when(cid == 0)
            def _():
                pltpu.sync_copy(ea_hbm.at[pl.ds(ebase, CH)], ea_v)
                pltpu.async_copy(xlo_hbm.at[ridx_v], msg_v, sem).wait()

            @pl.when(cid == 1)
            def _():
                pltpu.async_copy(xhi_hbm.at[ridx_v], msg_v, sem).wait()

            pltpu.sync_copy(msg_v, accx_s.at[cidx_v], add=True)

            @pl.when(cid == 0)
            def _():
                pltpu.sync_copy(ea_v, acc2_s.at[cidx_v], add=True)

            @pl.when(cid == 1)
            def _():
                pltpu.sync_copy(ones_v, acc2_s.at[cidx_v], add=True)

        @pl.loop(0, base_chunks)
        def _(i):
            do_chunk(i * NS + sid)

        if extra:
            @pl.when(sid < extra)
            def _():
                do_chunk(base_chunks * NS + sid)

        plsc.subcore_barrier()

        def out_rows(base, size):
            pltpu.sync_copy(accx_s.at[pl.ds(base, size)],
                            ox_hbm.at[cid, pl.ds(base, size)])
            pltpu.sync_copy(acc2_s.at[pl.ds(base, size)],
                            o2_hbm.at[cid, pl.ds(base, size)])

        out_rows(zbase, rpt)
        if rem:
            @pl.when(sid == NS - 1)
            def _():
                out_rows(NS * rpt, rem)

    return sc_kernel(xlo, xhi, row, col, eattr, zx, ze)


def _tc_body(x_ref, sxlo_ref, sxhi_ref, se_ref, cnt_ref,
             w_ref, b_ref, o_ref):
    D = x_ref.shape[1]
    DH = sxlo_ref.shape[2]
    DE = se_ref.shape[2]
    cnt = cnt_ref[0][:, 0:1]
    inv = 1.0 / jnp.maximum(cnt, 1.0)
    acc = jnp.dot(x_ref[...], w_ref[0:D, :], preferred_element_type=jnp.float32)
    agg = jnp.dot(sxlo_ref[0], w_ref[D:D + DH, :],
                  preferred_element_type=jnp.float32)
    agg = agg + jnp.dot(sxhi_ref[0], w_ref[D + DH:2 * D, :],
                        preferred_element_type=jnp.float32)
    agg = agg + jnp.dot(se_ref[0], w_ref[2 * D:2 * D + DE, :],
                        preferred_element_type=jnp.float32)
    o_ref[...] = jnp.maximum(acc + agg * inv + b_ref[...], 0.0)


def _tc_update(x, ox, o2, W, b):
    N, D = x.shape
    DH = ox.shape[2]
    DE = o2.shape[2]
    BN = 1000
    grid = (N // BN,)
    fan_in = W.shape[0]
    return pl.pallas_call(
        _tc_body,
        grid=grid,
        in_specs=[
            pl.BlockSpec((BN, D), lambda i: (i, 0)),
            pl.BlockSpec((1, BN, DH), lambda i: (0, i, 0)),
            pl.BlockSpec((1, BN, DH), lambda i: (1, i, 0)),
            pl.BlockSpec((1, BN, DE), lambda i: (0, i, 0)),
            pl.BlockSpec((1, BN, DE), lambda i: (1, i, 0)),
            pl.BlockSpec((fan_in, D), lambda i: (0, 0)),
            pl.BlockSpec((1, D), lambda i: (0, 0)),
        ],
        out_specs=pl.BlockSpec((BN, D), lambda i: (i, 0)),
        out_shape=jax.ShapeDtypeStruct((N, D), jnp.float32),
    )(x, ox, ox, o2, o2, W, b.reshape(1, D))


def kernel(x, edge_index, edge_attr, W, b):
    N, D = x.shape
    DH = D // 2
    DE = edge_attr.shape[1]
    row = edge_index[0]
    col = edge_index[1]
    xlo = x[:, :DH]
    xhi = x[:, DH:]
    zx = jnp.zeros((N, DH), jnp.float32)
    ze = jnp.zeros((N, DE), jnp.float32)
    ox, o2 = _sc_segment_sums(xlo, xhi, row, col, edge_attr, zx, ze)
    x_new = _tc_update(x, ox, o2, W, b)
    return (x_new, edge_attr)


# speedup vs baseline: 6.9789x; 1.7757x over previous
import functools

import jax
import jax.numpy as jnp
from jax import lax
from jax.experimental import pallas as pl
from jax.experimental.pallas import tpu as pltpu
from jax.experimental.pallas import tpu_sc as plsc

NC = 2
NS = 16
CH = 128
LANES = 16


def _sc_segment_sums(xs, row, col, eattr, zx, ze):
    N2, DH = xs.shape
    N = N2 // NC
    E, DE = eattr.shape
    n = E // CH
    base = n // NS
    extra = n - base * NS

    mesh = plsc.VectorSubcoreMesh(core_axis_name="c", subcore_axis_name="s")

    @functools.partial(
        pl.kernel,
        out_type=(
            jax.ShapeDtypeStruct((NC, N, DH), jnp.float32),
            jax.ShapeDtypeStruct((NC, N, DE), jnp.float32),
        ),
        mesh=mesh,
        compiler_params=pltpu.CompilerParams(use_tc_tiling_on_sc=False),
        scratch_types=[
            pltpu.VMEM((2, CH), jnp.int32),
            pltpu.VMEM((2, CH), jnp.int32),
            pltpu.VMEM((2, CH, DH), jnp.float32),
            pltpu.VMEM((2, CH, DE), jnp.float32),
            pltpu.SemaphoreType.DMA,
            pltpu.SemaphoreType.DMA,
            pltpu.SemaphoreType.DMA,
            pltpu.SemaphoreType.DMA,
            pltpu.SemaphoreType.DMA,
            pltpu.SemaphoreType.DMA,
            pltpu.SemaphoreType.DMA,
            pltpu.VMEM_SHARED((N, DH), jnp.float32),
            pltpu.VMEM_SHARED((N, DE), jnp.float32),
        ],
    )
    def sc_kernel(xs_hbm, row_hbm, col_hbm, ea_hbm, zx_hbm, ze_hbm,
                  ox_hbm, o2_hbm,
                  ridx_v, cidx_v, msg_v, ea_v,
                  sI0, sI1, sE0, sE1, sG0, sG1, sZ,
                  accx_s, acc2_s):
        cid = lax.axis_index("c")
        sid = lax.axis_index("s")
        sI = (sI0, sI1)
        sE = (sE0, sE1)
        sG = (sG0, sG1)
        idx_off = cid * N

        @pl.loop(0, CH)
        def _(i):
            ea_v[0, i, :] = jnp.full((DE,), 1.0, dtype=jnp.float32)
            ea_v[1, i, :] = jnp.full((DE,), 1.0, dtype=jnp.float32)

        rpt = (N // NS) // 8 * 8
        rem = N - NS * rpt
        zbase = sid * rpt

        def zero_rows(b, size):
            pltpu.async_copy(zx_hbm.at[pl.ds(b, size)],
                             accx_s.at[pl.ds(b, size)], sZ).wait()
            pltpu.async_copy(ze_hbm.at[pl.ds(b, size)],
                             acc2_s.at[pl.ds(b, size)], sZ).wait()

        def ebase(i):
            return (i * NS + sid) * CH

        def idx_start(i, b):
            eb = ebase(i)
            pltpu.make_async_copy(row_hbm.at[pl.ds(eb, CH)],
                                  ridx_v.at[b], sI[b]).start()
            pltpu.make_async_copy(col_hbm.at[pl.ds(eb, CH)],
                                  cidx_v.at[b], sI[b]).start()

        def idx_wait(b):
            pltpu.make_async_copy(row_hbm.at[pl.ds(0, CH)],
                                  ridx_v.at[b], sI[b]).wait()
            pltpu.make_async_copy(col_hbm.at[pl.ds(0, CH)],
                                  cidx_v.at[b], sI[b]).wait()
            for j in range(CH // LANES):
                sl = pl.ds(j * LANES, LANES)
                ridx_v[b, sl] = ridx_v[b, sl] + idx_off

        def ea_start(i, b):
            @pl.when(cid == 0)
            def _():
                pltpu.make_async_copy(ea_hbm.at[pl.ds(ebase(i), CH)],
                                      ea_v.at[b], sE[b]).start()

        def ea_wait(b):
            @pl.when(cid == 0)
            def _():
                pltpu.make_async_copy(ea_hbm.at[pl.ds(0, CH)],
                                      ea_v.at[b], sE[b]).wait()

        def gather_start(b):
            pltpu.make_async_copy(xs_hbm.at[ridx_v.at[b]],
                                  msg_v.at[b], sG[b]).start()

        def gather_wait(b):
            pltpu.make_async_copy(xs_hbm.at[ridx_v.at[b]],
                                  msg_v.at[b], sG[b]).wait()

        def scatter(b):
            pltpu.sync_copy(msg_v.at[b], accx_s.at[cidx_v.at[b]], add=True)
            pltpu.sync_copy(ea_v.at[b], acc2_s.at[cidx_v.at[b]], add=True)

        def run_chunks():
            idx_start(0, 0)
            ea_start(0, 0)
            idx_start(1, 1)
            ea_start(1, 1)
            idx_wait(0)
            gather_start(0)

            @pl.loop(0, (base - 2) // 2)
            def _(k):
                for b in range(2):
                    i = k * 2 + b
                    idx_wait(1 - b)
                    gather_start(1 - b)
                    gather_wait(b)
                    ea_wait(b)
                    scatter(b)
                    idx_start(i + 2, b)
                    ea_start(i + 2, b)

            idx_wait(1)
            gather_start(1)
            gather_wait(0)
            ea_wait(0)
            scatter(0)
            gather_wait(1)
            ea_wait(1)
            scatter(1)

            if extra:
                @pl.when(sid < extra)
                def _():
                    eb = (base * NS + sid) * CH
                    pltpu.make_async_copy(row_hbm.at[pl.ds(eb, CH)],
                                          ridx_v.at[0], sI[0]).start()
                    pltpu.make_async_copy(col_hbm.at[pl.ds(eb, CH)],
                                          cidx_v.at[0], sI[0]).start()
                    @pl.when(cid == 0)
                    def _():
                        pltpu.make_async_copy(ea_hbm.at[pl.ds(eb, CH)],
                                              ea_v.at[0], sE[0]).start()
                    idx_wait(0)
                    gather_start(0)
                    gather_wait(0)
                    ea_wait(0)
                    scatter(0)

        def out_rows(b, size):
            pltpu.sync_copy(accx_s.at[pl.ds(b, size)],
                            ox_hbm.at[cid, pl.ds(b, size)])
            pltpu.sync_copy(acc2_s.at[pl.ds(b, size)],
                            o2_hbm.at[cid, pl.ds(b, size)])

        zero_rows(zbase, rpt)
        if rem:
            @pl.when(sid == NS - 1)
            def _():
                zero_rows(NS * rpt, rem)
        plsc.subcore_barrier()
        run_chunks()
        plsc.subcore_barrier()
        out_rows(zbase, rpt)
        if rem:
            @pl.when(sid == NS - 1)
            def _():
                out_rows(NS * rpt, rem)

    return sc_kernel(xs, row, col, eattr, zx, ze)


def _tc_body(x_ref, sxlo_ref, sxhi_ref, se_ref, cnt_ref,
             w_ref, b_ref, o_ref):
    D = x_ref.shape[1]
    DH = sxlo_ref.shape[2]
    DE = se_ref.shape[2]
    cnt = cnt_ref[0][:, 0:1]
    inv = 1.0 / jnp.maximum(cnt, 1.0)
    acc = jnp.dot(x_ref[...], w_ref[0:D, :], preferred_element_type=jnp.float32)
    agg = jnp.dot(sxlo_ref[0], w_ref[D:D + DH, :],
                  preferred_element_type=jnp.float32)
    agg = agg + jnp.dot(sxhi_ref[0], w_ref[D + DH:2 * D, :],
                        preferred_element_type=jnp.float32)
    agg = agg + jnp.dot(se_ref[0], w_ref[2 * D:2 * D + DE, :],
                        preferred_element_type=jnp.float32)
    o_ref[...] = jnp.maximum(acc + agg * inv + b_ref[...], 0.0)


def _tc_update(x, ox, o2, W, b):
    N, D = x.shape
    DH = ox.shape[2]
    DE = o2.shape[2]
    BN = 1000
    grid = (N // BN,)
    fan_in = W.shape[0]
    return pl.pallas_call(
        _tc_body,
        grid=grid,
        in_specs=[
            pl.BlockSpec((BN, D), lambda i: (i, 0)),
            pl.BlockSpec((1, BN, DH), lambda i: (0, i, 0)),
            pl.BlockSpec((1, BN, DH), lambda i: (1, i, 0)),
            pl.BlockSpec((1, BN, DE), lambda i: (0, i, 0)),
            pl.BlockSpec((1, BN, DE), lambda i: (1, i, 0)),
            pl.BlockSpec((fan_in, D), lambda i: (0, 0)),
            pl.BlockSpec((1, D), lambda i: (0, 0)),
        ],
        out_specs=pl.BlockSpec((BN, D), lambda i: (i, 0)),
        out_shape=jax.ShapeDtypeStruct((N, D), jnp.float32),
    )(x, ox, ox, o2, o2, W, b.reshape(1, D))


def kernel(x, edge_index, edge_attr, W, b):
    N, D = x.shape
    DH = D // 2
    DE = edge_attr.shape[1]
    row = edge_index[0]
    col = edge_index[1]
    xs = jnp.concatenate([x[:, :DH], x[:, DH:]], axis=0)
    zx = jnp.zeros((N, DH), jnp.float32)
    ze = jnp.zeros((N, DE), jnp.float32)
    ox, o2 = _sc_segment_sums(xs, row, col, edge_attr, zx, ze)
    x_new = _tc_update(x, ox, o2, W, b)
    return (x_new, edge_attr)


# speedup vs baseline: 8.2777x; 1.1861x over previous
import functools

import jax
import jax.numpy as jnp
from jax import lax
from jax.experimental import pallas as pl
from jax.experimental.pallas import tpu as pltpu
from jax.experimental.pallas import tpu_sc as plsc

NC = 2
NS = 16
CH = 128
LANES = 16
NB = 4


def _sc_segment_sums(xs, row, col, eattr, zx, ze):
    N2, DH = xs.shape
    N = N2 // NC
    E, DE = eattr.shape
    n = E // CH
    base = n // NS
    extra = n - base * NS
    assert base % NB == 0 and base >= 3 * NB

    mesh = plsc.VectorSubcoreMesh(core_axis_name="c", subcore_axis_name="s")

    @functools.partial(
        pl.kernel,
        out_type=(
            jax.ShapeDtypeStruct((NC, N, DH), jnp.float32),
            jax.ShapeDtypeStruct((NC, N, DE), jnp.float32),
        ),
        mesh=mesh,
        compiler_params=pltpu.CompilerParams(use_tc_tiling_on_sc=False),
        scratch_types=[
            pltpu.VMEM((NB, CH), jnp.int32),
            pltpu.VMEM((NB, CH), jnp.int32),
            pltpu.VMEM((NB, CH, DH), jnp.float32),
            pltpu.VMEM((NB, CH, DE), jnp.float32),
            pltpu.VMEM_SHARED((N, DH), jnp.float32),
            pltpu.VMEM_SHARED((N, DE), jnp.float32),
        ] + [pltpu.SemaphoreType.DMA] * (4 * NB + 1),
    )
    def sc_kernel(xs_hbm, row_hbm, col_hbm, ea_hbm, zx_hbm, ze_hbm,
                  ox_hbm, o2_hbm,
                  ridx_v, cidx_v, msg_v, ea_v, accx_s, acc2_s,
                  *sems):
        cid = lax.axis_index("c")
        sid = lax.axis_index("s")
        sI = sems[0:NB]
        sE = sems[NB:2 * NB]
        sG = sems[2 * NB:3 * NB]
        sS = sems[3 * NB:4 * NB]
        sZ = sems[4 * NB]
        idx_off = cid * N

        @pl.loop(0, CH)
        def _(i):
            for b in range(NB):
                ea_v[b, i, :] = jnp.full((DE,), 1.0, dtype=jnp.float32)

        rpt = (N // NS) // 8 * 8
        rem = N - NS * rpt
        zbase = sid * rpt

        def zero_rows(rb, size):
            pltpu.async_copy(zx_hbm.at[pl.ds(rb, size)],
                             accx_s.at[pl.ds(rb, size)], sZ).wait()
            pltpu.async_copy(ze_hbm.at[pl.ds(rb, size)],
                             acc2_s.at[pl.ds(rb, size)], sZ).wait()

        def ebase(i):
            return (i * NS + sid) * CH

        def idx_start(i, b):
            eb = ebase(i)
            pltpu.async_copy(row_hbm.at[pl.ds(eb, CH)], ridx_v.at[b], sI[b])
            pltpu.async_copy(col_hbm.at[pl.ds(eb, CH)], cidx_v.at[b], sI[b])

            @pl.when(cid == 0)
            def _():
                pltpu.async_copy(ea_hbm.at[pl.ds(eb, CH)], ea_v.at[b], sE[b])

        def stage_a(i, b):
            pltpu.make_async_copy(row_hbm.at[pl.ds(0, CH)],
                                  ridx_v.at[b], sI[b]).wait()
            pltpu.make_async_copy(col_hbm.at[pl.ds(0, CH)],
                                  cidx_v.at[b], sI[b]).wait()
            for j in range(CH // LANES):
                sl = pl.ds(j * LANES, LANES)
                ridx_v[b, sl] = ridx_v[b, sl] + idx_off
            pltpu.async_copy(xs_hbm.at[ridx_v.at[b]], msg_v.at[b], sG[b])

        def stage_b(i, b):
            pltpu.make_async_copy(xs_hbm.at[ridx_v.at[b]],
                                  msg_v.at[b], sG[b]).wait()

            @pl.when(cid == 0)
            def _():
                pltpu.make_async_copy(ea_hbm.at[pl.ds(0, CH)],
                                      ea_v.at[b], sE[b]).wait()
            pltpu.async_copy(msg_v.at[b], accx_s.at[cidx_v.at[b]],
                             sS[b], add=True)
            pltpu.async_copy(ea_v.at[b], acc2_s.at[cidx_v.at[b]],
                             sS[b], add=True)

        def stage_c(i, b):
            pltpu.make_async_copy(msg_v.at[b],
                                  accx_s.at[cidx_v.at[b]], sS[b]).wait()
            pltpu.make_async_copy(ea_v.at[b],
                                  acc2_s.at[cidx_v.at[b]], sS[b]).wait()

        def run_chunks():
            for b in range(NB):
                idx_start(b, b)
            stage_a(0, 0)
            stage_a(1, 1)
            stage_b(0, 0)
            stage_a(2, 2)
            stage_b(1, 1)
            stage_c(0, 0)
            idx_start(NB, 0)
            stage_a(3, 3)
            stage_b(2, 2)
            stage_c(1, 1)
            idx_start(NB + 1, 1)

            @pl.loop(0, (base - 2 * NB) // NB)
            def _(k):
                for b in range(NB):
                    i = NB + k * NB + b
                    stage_a(i, b)
                    stage_b(i - 1, (b - 1) % NB)
                    stage_c(i - 2, (b - 2) % NB)
                    idx_start(i + 2, (b + 2) % NB)

            for i in range(base - NB, base):
                b = i % NB
                stage_a(i, b)
                stage_b(i - 1, (b - 1) % NB)
                stage_c(i - 2, (b - 2) % NB)
                if i + 2 < base:
                    idx_start(i + 2, (b + 2) % NB)
            stage_b(base - 1, (base - 1) % NB)
            stage_c(base - 2, (base - 2) % NB)
            stage_c(base - 1, (base - 1) % NB)

            if extra:
                @pl.when(sid < extra)
                def _():
                    idx_start(base, 0)
                    stage_a(base, 0)
                    stage_b(base, 0)
                    stage_c(base, 0)

        def out_rows(rb, size):
            pltpu.sync_copy(accx_s.at[pl.ds(rb, size)],
                            ox_hbm.at[cid, pl.ds(rb, size)])
            pltpu.sync_copy(acc2_s.at[pl.ds(rb, size)],
                            o2_hbm.at[cid, pl.ds(rb, size)])

        zero_rows(zbase, rpt)
        if rem:
            @pl.when(sid == NS - 1)
            def _():
                zero_rows(NS * rpt, rem)
        plsc.subcore_barrier()
        run_chunks()
        plsc.subcore_barrier()
        out_rows(zbase, rpt)
        if rem:
            @pl.when(sid == NS - 1)
            def _():
                out_rows(NS * rpt, rem)

    return sc_kernel(xs, row, col, eattr, zx, ze)


def _tc_body(x_ref, sxlo_ref, sxhi_ref, se_ref, cnt_ref,
             w_ref, b_ref, o_ref):
    D = x_ref.shape[1]
    DH = sxlo_ref.shape[2]
    DE = se_ref.shape[2]
    cnt = cnt_ref[0][:, 0:1]
    inv = 1.0 / jnp.maximum(cnt, 1.0)
    acc = jnp.dot(x_ref[...], w_ref[0:D, :], preferred_element_type=jnp.float32)
    agg = jnp.dot(sxlo_ref[0], w_ref[D:D + DH, :],
                  preferred_element_type=jnp.float32)
    agg = agg + jnp.dot(sxhi_ref[0], w_ref[D + DH:2 * D, :],
                        preferred_element_type=jnp.float32)
    agg = agg + jnp.dot(se_ref[0], w_ref[2 * D:2 * D + DE, :],
                        preferred_element_type=jnp.float32)
    o_ref[...] = jnp.maximum(acc + agg * inv + b_ref[...], 0.0)


def _tc_update(x, ox, o2, W, b):
    N, D = x.shape
    DH = ox.shape[2]
    DE = o2.shape[2]
    BN = 1000
    grid = (N // BN,)
    fan_in = W.shape[0]
    return pl.pallas_call(
        _tc_body,
        grid=grid,
        in_specs=[
            pl.BlockSpec((BN, D), lambda i: (i, 0)),
            pl.BlockSpec((1, BN, DH), lambda i: (0, i, 0)),
            pl.BlockSpec((1, BN, DH), lambda i: (1, i, 0)),
            pl.BlockSpec((1, BN, DE), lambda i: (0, i, 0)),
            pl.BlockSpec((1, BN, DE), lambda i: (1, i, 0)),
            pl.BlockSpec((fan_in, D), lambda i: (0, 0)),
            pl.BlockSpec((1, D), lambda i: (0, 0)),
        ],
        out_specs=pl.BlockSpec((BN, D), lambda i: (i, 0)),
        out_shape=jax.ShapeDtypeStruct((N, D), jnp.float32),
    )(x, ox, ox, o2, o2, W, b.reshape(1, D))


def kernel(x, edge_index, edge_attr, W, b):
    N, D = x.shape
    DH = D // 2
    DE = edge_attr.shape[1]
    row = edge_index[0]
    col = edge_index[1]
    xs = jnp.concatenate([x[:, :DH], x[:, DH:]], axis=0)
    zx = jnp.zeros((N, DH), jnp.float32)
    ze = jnp.zeros((N, DE), jnp.float32)
    ox, o2 = _sc_segment_sums(xs, row, col, edge_attr, zx, ze)
    x_new = _tc_update(x, ox, o2, W, b)
    return (x_new, edge_attr)


# speedup vs baseline: 8.8300x; 1.0667x over previous
import functools

import jax
import jax.numpy as jnp
from jax import lax
from jax.experimental import pallas as pl
from jax.experimental.pallas import tpu as pltpu
from jax.experimental.pallas import tpu_sc as plsc

NC = 2
NS = 16
CH = 128
LANES = 16
NB = 4


def _sc_segment_sums(x2, row, col, eattr, zx, ze):
    N2, DH = x2.shape
    N = N2 // NC
    E, DE = eattr.shape
    n = E // CH
    base = n // NS
    extra = n - base * NS
    assert base % NB == 0 and base >= 3 * NB

    mesh = plsc.VectorSubcoreMesh(core_axis_name="c", subcore_axis_name="s")

    @functools.partial(
        pl.kernel,
        out_type=(
            jax.ShapeDtypeStruct((NC, N, DH), jnp.float32),
            jax.ShapeDtypeStruct((NC, N, DE), jnp.float32),
        ),
        mesh=mesh,
        compiler_params=pltpu.CompilerParams(use_tc_tiling_on_sc=False),
        scratch_types=[
            pltpu.VMEM((NB, CH), jnp.int32),
            pltpu.VMEM((NB, CH), jnp.int32),
            pltpu.VMEM((NB, CH, DH), jnp.float32),
            pltpu.VMEM((NB, CH, DE), jnp.float32),
            pltpu.VMEM_SHARED((N, DH), jnp.float32),
            pltpu.VMEM_SHARED((N, DE), jnp.float32),
        ] + [pltpu.SemaphoreType.DMA] * (4 * NB + 1),
    )
    def sc_kernel(x2_hbm, row_hbm, col_hbm, ea_hbm, zx_hbm, ze_hbm,
                  ox_hbm, o2_hbm,
                  ridx_v, cidx_v, msg_v, ea_v, accx_s, acc2_s,
                  *sems):
        cid = lax.axis_index("c")
        sid = lax.axis_index("s")
        sI = sems[0:NB]
        sE = sems[NB:2 * NB]
        sG = sems[2 * NB:3 * NB]
        sS = sems[3 * NB:4 * NB]
        sZ = sems[4 * NB]

        @pl.loop(0, CH)
        def _(i):
            for b in range(NB):
                ea_v[b, i, :] = jnp.full((DE,), 1.0, dtype=jnp.float32)

        rpt = (N // NS) // 8 * 8
        rem = N - NS * rpt
        zbase = sid * rpt

        def zero_rows(rb, size):
            pltpu.async_copy(zx_hbm.at[pl.ds(rb, size)],
                             accx_s.at[pl.ds(rb, size)], sZ).wait()
            pltpu.async_copy(ze_hbm.at[pl.ds(rb, size)],
                             acc2_s.at[pl.ds(rb, size)], sZ).wait()

        def ebase(i):
            return (i * NS + sid) * CH

        def idx_start(i, b):
            eb = ebase(i)
            pltpu.async_copy(row_hbm.at[pl.ds(eb, CH)], ridx_v.at[b], sI[b])
            pltpu.async_copy(col_hbm.at[pl.ds(eb, CH)], cidx_v.at[b], sI[b])

            @pl.when(cid == 0)
            def _():
                pltpu.async_copy(ea_hbm.at[pl.ds(eb, CH)], ea_v.at[b], sE[b])

        def stage_a(b):
            pltpu.make_async_copy(row_hbm.at[pl.ds(0, CH)],
                                  ridx_v.at[b], sI[b]).wait()
            pltpu.make_async_copy(col_hbm.at[pl.ds(0, CH)],
                                  cidx_v.at[b], sI[b]).wait()
            for j in range(CH // LANES):
                sl = pl.ds(j * LANES, LANES)
                ridx_v[b, sl] = ridx_v[b, sl] * 2 + cid
            pltpu.async_copy(x2_hbm.at[ridx_v.at[b]], msg_v.at[b], sG[b])

        def stage_b(b):
            pltpu.make_async_copy(x2_hbm.at[ridx_v.at[b]],
                                  msg_v.at[b], sG[b]).wait()

            @pl.when(cid == 0)
            def _():
                pltpu.make_async_copy(ea_hbm.at[pl.ds(0, CH)],
                                      ea_v.at[b], sE[b]).wait()
            pltpu.async_copy(msg_v.at[b], accx_s.at[cidx_v.at[b]],
                             sS[b], add=True)
            pltpu.async_copy(ea_v.at[b], acc2_s.at[cidx_v.at[b]],
                             sS[b], add=True)

        def stage_c(b):
            pltpu.make_async_copy(msg_v.at[b],
                                  accx_s.at[cidx_v.at[b]], sS[b]).wait()
            pltpu.make_async_copy(ea_v.at[b],
                                  acc2_s.at[cidx_v.at[b]], sS[b]).wait()

        def run_chunks():
            for b in range(NB):
                idx_start(b, b)
            stage_a(0)
            stage_a(1)
            stage_b(0)
            stage_a(2)
            stage_b(1)
            stage_c(0)
            idx_start(NB, 0)
            stage_a(3)
            stage_b(2)
            stage_c(1)
            idx_start(NB + 1, 1)

            @pl.loop(0, (base - 2 * NB) // NB)
            def _(k):
                for b in range(NB):
                    i = NB + k * NB + b
                    stage_a(b)
                    stage_b((b - 1) % NB)
                    stage_c((b - 2) % NB)
                    idx_start(i + 2, (b + 2) % NB)

            for i in range(base - NB, base):
                b = i % NB
                stage_a(b)
                stage_b((b - 1) % NB)
                stage_c((b - 2) % NB)
                if i + 2 < base:
                    idx_start(i + 2, (b + 2) % NB)
            stage_b((base - 1) % NB)
            stage_c((base - 2) % NB)
            stage_c((base - 1) % NB)

            if extra:
                @pl.when(sid < extra)
                def _():
                    idx_start(base, 0)
                    stage_a(0)
                    stage_b(0)
                    stage_c(0)

        def out_rows(rb, size):
            pltpu.sync_copy(accx_s.at[pl.ds(rb, size)],
                            ox_hbm.at[cid, pl.ds(rb, size)])
            pltpu.sync_copy(acc2_s.at[pl.ds(rb, size)],
                            o2_hbm.at[cid, pl.ds(rb, size)])

        zero_rows(zbase, rpt)
        if rem:
            @pl.when(sid == NS - 1)
            def _():
                zero_rows(NS * rpt, rem)
        plsc.subcore_barrier()
        run_chunks()
        plsc.subcore_barrier()
        out_rows(zbase, rpt)
        if rem:
            @pl.when(sid == NS - 1)
            def _():
                out_rows(NS * rpt, rem)

    return sc_kernel(x2, row, col, eattr, zx, ze)


def _tc_body(x_ref, sxlo_ref, sxhi_ref, se_ref, cnt_ref,
             w_ref, b_ref, o_ref):
    D = x_ref.shape[1]
    DH = sxlo_ref.shape[2]
    DE = se_ref.shape[2]
    cnt = cnt_ref[0][:, 0:1]
    inv = 1.0 / jnp.maximum(cnt, 1.0)
    acc = jnp.dot(x_ref[...], w_ref[0:D, :], preferred_element_type=jnp.float32)
    agg = jnp.dot(sxlo_ref[0], w_ref[D:D + DH, :],
                  preferred_element_type=jnp.float32)
    agg = agg + jnp.dot(sxhi_ref[0], w_ref[D + DH:2 * D, :],
                        preferred_element_type=jnp.float32)
    agg = agg + jnp.dot(se_ref[0], w_ref[2 * D:2 * D + DE, :],
                        preferred_element_type=jnp.float32)
    o_ref[...] = jnp.maximum(acc + agg * inv + b_ref[...], 0.0)


def _tc_update(x, ox, o2, W, b):
    N, D = x.shape
    DH = ox.shape[2]
    DE = o2.shape[2]
    BN = 1000
    grid = (N // BN,)
    fan_in = W.shape[0]
    return pl.pallas_call(
        _tc_body,
        grid=grid,
        in_specs=[
            pl.BlockSpec((BN, D), lambda i: (i, 0)),
            pl.BlockSpec((1, BN, DH), lambda i: (0, i, 0)),
            pl.BlockSpec((1, BN, DH), lambda i: (1, i, 0)),
            pl.BlockSpec((1, BN, DE), lambda i: (0, i, 0)),
            pl.BlockSpec((1, BN, DE), lambda i: (1, i, 0)),
            pl.BlockSpec((fan_in, D), lambda i: (0, 0)),
            pl.BlockSpec((1, D), lambda i: (0, 0)),
        ],
        out_specs=pl.BlockSpec((BN, D), lambda i: (i, 0)),
        out_shape=jax.ShapeDtypeStruct((N, D), jnp.float32),
    )(x, ox, ox, o2, o2, W, b.reshape(1, D))


def kernel(x, edge_index, edge_attr, W, b):
    N, D = x.shape
    DH = D // 2
    DE = edge_attr.shape[1]
    row = edge_index[0]
    col = edge_index[1]
    x2 = x.reshape(NC * N, DH)
    zx = jnp.zeros((N, DH), jnp.float32)
    ze = jnp.zeros((N, DE), jnp.float32)
    ox, o2 = _sc_segment_sums(x2, row, col, edge_attr, zx, ze)
    x_new = _tc_update(x, ox, o2, W, b)
    return (x_new, edge_attr)
